# baseline (speedup 1.0000x reference)
import jax
import jax.numpy as jnp
from jax import lax
from jax.experimental import pallas as pl
from jax.experimental.pallas import tpu as pltpu
from jax.experimental.pallas import tpu_sc as plsc

B = 64
S = 2048
E = 128

NC = 2
NS = 16
NW = NC * NS

PG = 8
BG = NW // PG
BATCH_PER_G = B // BG
POS_PER_P = S // PG

CHUNK = 128
CH_PER_B = POS_PER_P // CHUNK
NCH = BATCH_PER_G * CH_PER_B
NBUF = 6
LOOK = 4


def _tpe_body(x_hbm, tok_hbm, pos_hbm, out_hbm, idx_all, rows_v, pos_sh,
              *sems):
    sid = lax.axis_index("s")
    wid = sid * NC + lax.axis_index("c")
    g = wid // PG
    p = wid % PG
    pos_base = p * POS_PER_P
    idx_sem = sems[0]
    gat_sems = sems[1:1 + NBUF]
    st_sems = sems[1 + NBUF:]
    mine = pos_sh.at[pl.ds(pos_base, POS_PER_P)]

    for b in range(BATCH_PER_G):
        batch = g * BATCH_PER_G + b
        pltpu.async_copy(x_hbm.at[batch].at[pl.ds(p * CH_PER_B, CH_PER_B)],
                         idx_all.at[pl.ds(b * CH_PER_B, CH_PER_B)], idx_sem)
    pltpu.sync_copy(pos_hbm.at[pl.ds(pos_base, POS_PER_P)], mine)
    for b in range(BATCH_PER_G):
        batch = g * BATCH_PER_G + b
        pltpu.make_async_copy(
            x_hbm.at[batch].at[pl.ds(p * CH_PER_B, CH_PER_B)],
            idx_all.at[pl.ds(b * CH_PER_B, CH_PER_B)], idx_sem).wait()

    def out_slc(c):
        b, h = c // CH_PER_B, c % CH_PER_B
        batch = g * BATCH_PER_G + b
        return out_hbm.at[pl.ds(batch * S + pos_base + h * CHUNK, CHUNK)]

    def stage_a(c):
        r = c % NBUF
        buf = rows_v.at[r]
        if c >= NBUF:
            pltpu.make_async_copy(buf, out_slc(c - NBUF), st_sems[r]).wait()
        h = c % CH_PER_B
        pltpu.sync_copy(mine.at[pl.ds(h * CHUNK, CHUNK)], buf)
        pltpu.async_copy(tok_hbm.at[idx_all.at[c]], buf, gat_sems[r],
                         add=True)

    def stage_b(c):
        r = c % NBUF
        buf = rows_v.at[r]
        pltpu.make_async_copy(tok_hbm.at[idx_all.at[c]], buf,
                              gat_sems[r]).wait()
        pltpu.async_copy(buf, out_slc(c), st_sems[r])

    for c in range(LOOK):
        stage_a(c)
    for c in range(NCH):
        if c + LOOK < NCH:
            stage_a(c + LOOK)
        stage_b(c)

    for c in range(NCH - NBUF, NCH):
        r = c % NBUF
        pltpu.make_async_copy(rows_v.at[r], out_slc(c), st_sems[r]).wait()


def kernel(x, token_table, pos_table):
    xi = x.astype(jnp.int32).reshape(B, S // CHUNK, CHUNK)
    mesh = plsc.VectorSubcoreMesh(core_axis_name="c", subcore_axis_name="s")
    f = pl.kernel(
        _tpe_body,
        out_type=jax.ShapeDtypeStruct((B * S, E), jnp.float32),
        mesh=mesh,
        scratch_types=[
            pltpu.VMEM((NCH, CHUNK), jnp.int32),
            pltpu.VMEM((NBUF, CHUNK, E), jnp.float32),
            pltpu.VMEM_SHARED((S, E), jnp.float32),
        ] + [pltpu.SemaphoreType.DMA] * (1 + 2 * NBUF),
    )
    out = f(xi, token_table, pos_table)
    return out.reshape(B, S, E)
